# baseline (speedup 1.0000x reference)
import functools

import jax
import jax.numpy as jnp
from jax import lax
from jax.experimental import pallas as pl
from jax.experimental.pallas import tpu as pltpu
from jax.experimental.pallas import tpu_sc as plsc

S, B, D, F, E = 2048, 2, 1024, 4096, 8
N = S * B
TBLK = 256
NBLK = N // TBLK
FB = F // 2
T = 256
NT = N // T + E
NPAD = NT * T

NC, NS = 2, 16
NW = NC * NS
TPW = N // NW


def _router_body(x_ref, c_ref, idx_ref, rank_ref, bc_ref, carry_ref):
    i = pl.program_id(0)

    @pl.when(i == 0)
    def _():
        carry_ref[...] = jnp.zeros_like(carry_ref)

    xb = jnp.concatenate([x_ref[:, 0, :], x_ref[:, 1, :]], axis=0)
    ct = c_ref[...]
    logits = lax.dot_general(xb, ct, (((1,), (1,)), ((), ())),
                             preferred_element_type=jnp.float32)
    idx = jnp.argmax(logits, axis=-1).astype(jnp.int32)
    lanes = lax.broadcasted_iota(jnp.int32, (TBLK, E), 1)
    oh = (idx[:, None] == lanes).astype(jnp.float32)
    r_i = lax.broadcasted_iota(jnp.int32, (TBLK, TBLK), 0)
    c_i = lax.broadcasted_iota(jnp.int32, (TBLK, TBLK), 1)
    tri = (c_i < r_i).astype(jnp.float32)
    cum = jnp.dot(tri, oh, preferred_element_type=jnp.float32)
    rank = jnp.sum((cum + carry_ref[...]) * oh, axis=1)
    bc = jnp.sum(oh, axis=0)
    idx_ref[...] = idx.reshape(1, 1, TBLK)
    rank_ref[...] = rank.astype(jnp.int32).reshape(1, 1, TBLK)
    bc_ref[...] = bc.reshape(1, 1, E)
    carry_ref[...] = carry_ref[...] + bc[None, :]


def _run_router(x, centroids):
    return pl.pallas_call(
        _router_body,
        grid=(NBLK,),
        in_specs=[
            pl.BlockSpec((TBLK // B, B, D), lambda i: (i, 0, 0)),
            pl.BlockSpec((E, D), lambda i: (0, 0)),
        ],
        out_specs=[
            pl.BlockSpec((1, 1, TBLK), lambda i: (i, 0, 0)),
            pl.BlockSpec((1, 1, TBLK), lambda i: (i, 0, 0)),
            pl.BlockSpec((1, 1, E), lambda i: (i, 0, 0)),
        ],
        out_shape=[
            jax.ShapeDtypeStruct((NBLK, 1, TBLK), jnp.int32),
            jax.ShapeDtypeStruct((NBLK, 1, TBLK), jnp.int32),
            jax.ShapeDtypeStruct((NBLK, 1, E), jnp.float32),
        ],
        scratch_shapes=[pltpu.VMEM((1, E), jnp.float32)],
        compiler_params=pltpu.CompilerParams(
            dimension_semantics=("arbitrary",)),
    )(x, centroids)


def _routing_metadata(bc3):
    bc = bc3.reshape(NBLK, E).astype(jnp.int32)
    counts = jnp.sum(bc, axis=0)
    pc = ((counts + T - 1) // T) * T
    poff = jnp.cumsum(pc) - pc
    off16 = jnp.zeros((16,), jnp.int32).at[:E].set(poff.astype(jnp.int32))
    tend = (poff + pc) // T
    kk = jnp.arange(NT, dtype=jnp.int32)
    te = jnp.minimum(
        jnp.sum((kk[:, None] >= tend[None, :]).astype(jnp.int32), axis=1),
        E - 1).astype(jnp.int32)
    valid = (kk < tend[E - 1]).astype(jnp.int32)
    return off16, te, valid


def _dispatch_body(x_hbm, idx_hbm, rank_hbm, off_hbm, xs_hbm, slot_hbm,
                   idxv, rankv, offv, slotf, slot2, rowbuf, sem):
    wid = lax.axis_index("s") * NC + lax.axis_index("c")
    base = wid * TPW
    s0 = (wid // B) * TPW
    bcol = wid % B
    pltpu.sync_copy(idx_hbm.at[pl.ds(base, TPW)], idxv)
    pltpu.sync_copy(rank_hbm.at[pl.ds(base, TPW)], rankv)
    pltpu.sync_copy(off_hbm, offv)
    for g in range(TPW // 16):
        e = idxv[pl.ds(g * 16, 16)]
        rk = rankv[pl.ds(g * 16, 16)]
        off = plsc.load_gather(offv, [e])
        sl = off + rk
        slotf[pl.ds(g * 16, 16)] = sl
        slot2[g // 4, pl.ds((g % 4) * 16, 16)] = sl
    pltpu.sync_copy(slotf, slot_hbm.at[pl.ds(base, TPW)])
    for h in range(2):
        pltpu.sync_copy(x_hbm.at[pl.ds(s0 + h * 64, 64), bcol], rowbuf)
        pltpu.async_copy(rowbuf, xs_hbm.at[slot2.at[h]], sem).wait()


def _cache_bf16(i, te_ref, w1_ref, w2_ref, w1b_ref, w2b_ref):
    iprev = jnp.maximum(i - 1, 0)
    changed = jnp.logical_or(i == 0, te_ref[i] != te_ref[iprev])

    @pl.when(changed)
    def _():
        w1b_ref[...] = w1_ref[0].astype(jnp.bfloat16)
        w2b_ref[...] = w2_ref[0].astype(jnp.bfloat16)


def _ffn_a_body(te_ref, valid_ref, xs_ref, lns_ref, lnb_ref, w1_ref, b1_ref,
                w2_ref, b2_ref, ya_ref, lnx_ref, w1b_ref, w2b_ref):
    i = pl.program_id(0)

    @pl.when(valid_ref[i] != 0)
    def _():
        _cache_bf16(i, te_ref, w1_ref, w2_ref, w1b_ref, w2b_ref)
        xv = xs_ref[...]
        mu = jnp.mean(xv, axis=1, keepdims=True)
        var = jnp.mean(jnp.square(xv - mu), axis=1, keepdims=True)
        lnx = ((xv - mu) * lax.rsqrt(var + 1e-5) * lns_ref[0, 0]
               + lnb_ref[0, 0])
        lnx_ref[...] = lnx
        h = jnp.maximum(
            jnp.dot(lnx.astype(jnp.bfloat16), w1b_ref[...],
                    preferred_element_type=jnp.float32) + b1_ref[0, 0], 0.0)
        ya_ref[...] = xv + b2_ref[0, 0] + jnp.dot(
            h.astype(jnp.bfloat16), w2b_ref[...],
            preferred_element_type=jnp.float32)


def _ffn_b_body(te_ref, valid_ref, lnx_ref, ya_ref, w1_ref, b1_ref, w2_ref,
                out_ref, w1b_ref, w2b_ref):
    i = pl.program_id(0)

    @pl.when(valid_ref[i] != 0)
    def _():
        _cache_bf16(i, te_ref, w1_ref, w2_ref, w1b_ref, w2b_ref)
        h = jnp.maximum(
            jnp.dot(lnx_ref[...].astype(jnp.bfloat16), w1b_ref[...],
                    preferred_element_type=jnp.float32) + b1_ref[0, 0], 0.0)
        out_ref[...] = ya_ref[...] + jnp.dot(
            h.astype(jnp.bfloat16), w2b_ref[...],
            preferred_element_type=jnp.float32)


def _run_ffn(te, valid, xs, ln_scale, ln_bias, W1, b1, W2, b2):
    b1r = b1.reshape(E, 1, F)
    ya, lnxs = pl.pallas_call(
        _ffn_a_body,
        grid_spec=pltpu.PrefetchScalarGridSpec(
            num_scalar_prefetch=2,
            grid=(NT,),
            in_specs=[
                pl.BlockSpec((T, D), lambda i, te, v: (i, 0)),
                pl.BlockSpec((1, 1, D), lambda i, te, v: (te[i], 0, 0)),
                pl.BlockSpec((1, 1, D), lambda i, te, v: (te[i], 0, 0)),
                pl.BlockSpec((1, D, FB), lambda i, te, v: (te[i], 0, 0)),
                pl.BlockSpec((1, 1, FB), lambda i, te, v: (te[i], 0, 0)),
                pl.BlockSpec((1, FB, D), lambda i, te, v: (te[i], 0, 0)),
                pl.BlockSpec((1, 1, D), lambda i, te, v: (te[i], 0, 0)),
            ],
            out_specs=[
                pl.BlockSpec((T, D), lambda i, te, v: (i, 0)),
                pl.BlockSpec((T, D), lambda i, te, v: (i, 0)),
            ],
            scratch_shapes=[
                pltpu.VMEM((D, FB), jnp.bfloat16),
                pltpu.VMEM((FB, D), jnp.bfloat16),
            ],
        ),
        out_shape=[
            jax.ShapeDtypeStruct((NPAD, D), jnp.float32),
            jax.ShapeDtypeStruct((NPAD, D), jnp.float32),
        ],
        compiler_params=pltpu.CompilerParams(
            dimension_semantics=("arbitrary",)),
    )(te, valid, xs, ln_scale.reshape(E, 1, D), ln_bias.reshape(E, 1, D),
      W1, b1r, W2, b2.reshape(E, 1, D))
    return pl.pallas_call(
        _ffn_b_body,
        grid_spec=pltpu.PrefetchScalarGridSpec(
            num_scalar_prefetch=2,
            grid=(NT,),
            in_specs=[
                pl.BlockSpec((T, D), lambda i, te, v: (i, 0)),
                pl.BlockSpec((T, D), lambda i, te, v: (i, 0)),
                pl.BlockSpec((1, D, FB), lambda i, te, v: (te[i], 0, 1)),
                pl.BlockSpec((1, 1, FB), lambda i, te, v: (te[i], 0, 1)),
                pl.BlockSpec((1, FB, D), lambda i, te, v: (te[i], 1, 0)),
            ],
            out_specs=pl.BlockSpec((T, D), lambda i, te, v: (i, 0)),
            scratch_shapes=[
                pltpu.VMEM((D, FB), jnp.bfloat16),
                pltpu.VMEM((FB, D), jnp.bfloat16),
            ],
        ),
        out_shape=jax.ShapeDtypeStruct((NPAD, D), jnp.float32),
        compiler_params=pltpu.CompilerParams(
            dimension_semantics=("arbitrary",)),
    )(te, valid, lnxs, ya, W1, b1r, W2)


def _undispatch_body(ys_hbm, slot_hbm, out_hbm, slot2, rowbuf, sem):
    wid = lax.axis_index("s") * NC + lax.axis_index("c")
    base = wid * TPW
    s0 = (wid // B) * TPW
    bcol = wid % B
    for h in range(2):
        pltpu.sync_copy(slot_hbm.at[pl.ds(base + h * 64, 64)], slot2.at[h])
        pltpu.async_copy(ys_hbm.at[slot2.at[h]], rowbuf, sem).wait()
        pltpu.sync_copy(rowbuf, out_hbm.at[pl.ds(s0 + h * 64, 64), bcol])


@functools.lru_cache(maxsize=1)
def _sc_kernels():
    mesh = plsc.VectorSubcoreMesh(core_axis_name="c", subcore_axis_name="s",
                                  num_cores=NC, num_subcores=NS)
    dispatch = pl.kernel(
        _dispatch_body,
        mesh=mesh,
        compiler_params=pltpu.CompilerParams(needs_layout_passes=False),
        out_type=[
            jax.ShapeDtypeStruct((NPAD, D), jnp.float32),
            jax.ShapeDtypeStruct((N,), jnp.int32),
        ],
        scratch_types=[
            pltpu.VMEM((TPW,), jnp.int32),
            pltpu.VMEM((TPW,), jnp.int32),
            pltpu.VMEM((16,), jnp.int32),
            pltpu.VMEM((TPW,), jnp.int32),
            pltpu.VMEM((2, 64), jnp.int32),
            pltpu.VMEM((64, D), jnp.float32),
            pltpu.SemaphoreType.DMA,
        ],
    )
    undispatch = pl.kernel(
        _undispatch_body,
        mesh=mesh,
        out_type=jax.ShapeDtypeStruct((S, B, D), jnp.float32),
        scratch_types=[
            pltpu.VMEM((2, 64), jnp.int32),
            pltpu.VMEM((64, D), jnp.float32),
            pltpu.SemaphoreType.DMA,
        ],
    )
    return dispatch, undispatch


def kernel(x, centroids, ln_scale, ln_bias, W1, b1, W2, b2):
    idx3, rank3, bc3 = _run_router(x, centroids)
    off16, te, valid = _routing_metadata(bc3)
    idx_flat = idx3.reshape(N)
    rank_flat = rank3.reshape(N)
    dispatch, undispatch = _sc_kernels()
    xs, slot = dispatch(x, idx_flat, rank_flat, off16)
    ys = _run_ffn(te, valid, xs, ln_scale, ln_bias, W1, b1, W2, b2)
    return undispatch(ys, slot)


# speedup vs baseline: 1.0503x; 1.0503x over previous
import functools

import jax
import jax.numpy as jnp
from jax import lax
from jax.experimental import pallas as pl
from jax.experimental.pallas import tpu as pltpu
from jax.experimental.pallas import tpu_sc as plsc

S, B, D, F, E = 2048, 2, 1024, 4096, 8
N = S * B
TBLK = 256
NBLK = N // TBLK
FB = F // 2
T = 256
NT = N // T + E
NPAD = NT * T

NC, NS = 2, 16
NW = NC * NS
TPW = N // NW


def _router_body(x_ref, c_ref, idx_ref, rank_ref, bc_ref, carry_ref):
    i = pl.program_id(0)

    @pl.when(i == 0)
    def _():
        carry_ref[...] = jnp.zeros_like(carry_ref)

    xb = jnp.concatenate([x_ref[:, 0, :], x_ref[:, 1, :]], axis=0)
    ct = c_ref[...]
    logits = lax.dot_general(xb, ct, (((1,), (1,)), ((), ())),
                             preferred_element_type=jnp.float32)
    idx = jnp.argmax(logits, axis=-1).astype(jnp.int32)
    lanes = lax.broadcasted_iota(jnp.int32, (TBLK, E), 1)
    oh = (idx[:, None] == lanes).astype(jnp.float32)
    r_i = lax.broadcasted_iota(jnp.int32, (TBLK, TBLK), 0)
    c_i = lax.broadcasted_iota(jnp.int32, (TBLK, TBLK), 1)
    tri = (c_i < r_i).astype(jnp.float32)
    cum = jnp.dot(tri, oh, preferred_element_type=jnp.float32)
    rank = jnp.sum((cum + carry_ref[...]) * oh, axis=1)
    bc = jnp.sum(oh, axis=0)
    idx_ref[...] = idx.reshape(1, 1, TBLK)
    rank_ref[...] = rank.astype(jnp.int32).reshape(1, 1, TBLK)
    bc_ref[...] = bc.reshape(1, 1, E)
    carry_ref[...] = carry_ref[...] + bc[None, :]


def _run_router(x, centroids):
    return pl.pallas_call(
        _router_body,
        grid=(NBLK,),
        in_specs=[
            pl.BlockSpec((TBLK // B, B, D), lambda i: (i, 0, 0)),
            pl.BlockSpec((E, D), lambda i: (0, 0)),
        ],
        out_specs=[
            pl.BlockSpec((1, 1, TBLK), lambda i: (i, 0, 0)),
            pl.BlockSpec((1, 1, TBLK), lambda i: (i, 0, 0)),
            pl.BlockSpec((1, 1, E), lambda i: (i, 0, 0)),
        ],
        out_shape=[
            jax.ShapeDtypeStruct((NBLK, 1, TBLK), jnp.int32),
            jax.ShapeDtypeStruct((NBLK, 1, TBLK), jnp.int32),
            jax.ShapeDtypeStruct((NBLK, 1, E), jnp.float32),
        ],
        scratch_shapes=[pltpu.VMEM((1, E), jnp.float32)],
        compiler_params=pltpu.CompilerParams(
            dimension_semantics=("arbitrary",)),
    )(x, centroids)


def _routing_metadata(bc3):
    bc = bc3.reshape(NBLK, E).astype(jnp.int32)
    counts = jnp.sum(bc, axis=0)
    pc = ((counts + T - 1) // T) * T
    poff = jnp.cumsum(pc) - pc
    off16 = jnp.zeros((16,), jnp.int32).at[:E].set(poff.astype(jnp.int32))
    tend = (poff + pc) // T
    kk = jnp.arange(NT, dtype=jnp.int32)
    te = jnp.minimum(
        jnp.sum((kk[:, None] >= tend[None, :]).astype(jnp.int32), axis=1),
        E - 1).astype(jnp.int32)
    valid = (kk < tend[E - 1]).astype(jnp.int32)
    return off16, te, valid


def _dispatch_body(x_hbm, idx_hbm, rank_hbm, off_hbm, xs_hbm, slot_hbm,
                   idxv, rankv, offv, slotf, slot2, rowbuf, sem):
    wid = lax.axis_index("s") * NC + lax.axis_index("c")
    base = wid * TPW
    s0 = (wid // B) * TPW
    bcol = wid % B
    pltpu.sync_copy(idx_hbm.at[pl.ds(base, TPW)], idxv)
    pltpu.sync_copy(rank_hbm.at[pl.ds(base, TPW)], rankv)
    pltpu.sync_copy(off_hbm, offv)
    for g in range(TPW // 16):
        e = idxv[pl.ds(g * 16, 16)]
        rk = rankv[pl.ds(g * 16, 16)]
        off = plsc.load_gather(offv, [e])
        sl = off + rk
        slotf[pl.ds(g * 16, 16)] = sl
        slot2[g // 4, pl.ds((g % 4) * 16, 16)] = sl
    pltpu.sync_copy(slotf, slot_hbm.at[pl.ds(base, TPW)])
    for h in range(2):
        pltpu.sync_copy(x_hbm.at[pl.ds(s0 + h * 64, 64), bcol], rowbuf)
        pltpu.async_copy(rowbuf, xs_hbm.at[slot2.at[h]], sem).wait()


def _ffn_a_body(te_ref, valid_ref, xs_ref, lns_ref, lnb_ref, w1_ref, b1_ref,
                w2_ref, b2_ref, ya_ref, lnx_ref):
    i = pl.program_id(0)

    @pl.when(valid_ref[i] != 0)
    def _():
        xv = xs_ref[...]
        mu = jnp.mean(xv, axis=1, keepdims=True)
        var = jnp.mean(jnp.square(xv - mu), axis=1, keepdims=True)
        lnx = ((xv - mu) * lax.rsqrt(var + 1e-5) * lns_ref[0, 0]
               + lnb_ref[0, 0])
        lnx_ref[...] = lnx
        h = jnp.maximum(
            jnp.dot(lnx, w1_ref[0], preferred_element_type=jnp.float32)
            + b1_ref[0, 0], 0.0)
        ya_ref[...] = xv + b2_ref[0, 0] + jnp.dot(
            h, w2_ref[0], preferred_element_type=jnp.float32)


def _ffn_b_body(te_ref, valid_ref, lnx_ref, ya_ref, w1_ref, b1_ref, w2_ref,
                out_ref):
    i = pl.program_id(0)

    @pl.when(valid_ref[i] != 0)
    def _():
        h = jnp.maximum(
            jnp.dot(lnx_ref[...], w1_ref[0],
                    preferred_element_type=jnp.float32) + b1_ref[0, 0], 0.0)
        out_ref[...] = ya_ref[...] + jnp.dot(
            h, w2_ref[0], preferred_element_type=jnp.float32)


def _run_ffn(te, valid, xs, ln_scale, ln_bias, W1, b1, W2, b2):
    b1r = b1.reshape(E, 1, F)
    ya, lnxs = pl.pallas_call(
        _ffn_a_body,
        grid_spec=pltpu.PrefetchScalarGridSpec(
            num_scalar_prefetch=2,
            grid=(NT,),
            in_specs=[
                pl.BlockSpec((T, D), lambda i, te, v: (i, 0)),
                pl.BlockSpec((1, 1, D), lambda i, te, v: (te[i], 0, 0)),
                pl.BlockSpec((1, 1, D), lambda i, te, v: (te[i], 0, 0)),
                pl.BlockSpec((1, D, FB), lambda i, te, v: (te[i], 0, 0)),
                pl.BlockSpec((1, 1, FB), lambda i, te, v: (te[i], 0, 0)),
                pl.BlockSpec((1, FB, D), lambda i, te, v: (te[i], 0, 0)),
                pl.BlockSpec((1, 1, D), lambda i, te, v: (te[i], 0, 0)),
            ],
            out_specs=[
                pl.BlockSpec((T, D), lambda i, te, v: (i, 0)),
                pl.BlockSpec((T, D), lambda i, te, v: (i, 0)),
            ],
        ),
        out_shape=[
            jax.ShapeDtypeStruct((NPAD, D), jnp.float32),
            jax.ShapeDtypeStruct((NPAD, D), jnp.float32),
        ],
        compiler_params=pltpu.CompilerParams(
            dimension_semantics=("arbitrary",)),
    )(te, valid, xs, ln_scale.reshape(E, 1, D), ln_bias.reshape(E, 1, D),
      W1, b1r, W2, b2.reshape(E, 1, D))
    return pl.pallas_call(
        _ffn_b_body,
        grid_spec=pltpu.PrefetchScalarGridSpec(
            num_scalar_prefetch=2,
            grid=(NT,),
            in_specs=[
                pl.BlockSpec((T, D), lambda i, te, v: (i, 0)),
                pl.BlockSpec((T, D), lambda i, te, v: (i, 0)),
                pl.BlockSpec((1, D, FB), lambda i, te, v: (te[i], 0, 1)),
                pl.BlockSpec((1, 1, FB), lambda i, te, v: (te[i], 0, 1)),
                pl.BlockSpec((1, FB, D), lambda i, te, v: (te[i], 1, 0)),
            ],
            out_specs=pl.BlockSpec((T, D), lambda i, te, v: (i, 0)),
        ),
        out_shape=jax.ShapeDtypeStruct((NPAD, D), jnp.float32),
        compiler_params=pltpu.CompilerParams(
            dimension_semantics=("arbitrary",)),
    )(te, valid, lnxs, ya, W1, b1r, W2)


def _undispatch_body(ys_hbm, slot_hbm, out_hbm, slot2, rowbuf, sem):
    wid = lax.axis_index("s") * NC + lax.axis_index("c")
    base = wid * TPW
    s0 = (wid // B) * TPW
    bcol = wid % B
    for h in range(2):
        pltpu.sync_copy(slot_hbm.at[pl.ds(base + h * 64, 64)], slot2.at[h])
        pltpu.async_copy(ys_hbm.at[slot2.at[h]], rowbuf, sem).wait()
        pltpu.sync_copy(rowbuf, out_hbm.at[pl.ds(s0 + h * 64, 64), bcol])


@functools.lru_cache(maxsize=1)
def _sc_kernels():
    mesh = plsc.VectorSubcoreMesh(core_axis_name="c", subcore_axis_name="s",
                                  num_cores=NC, num_subcores=NS)
    dispatch = pl.kernel(
        _dispatch_body,
        mesh=mesh,
        compiler_params=pltpu.CompilerParams(needs_layout_passes=False),
        out_type=[
            jax.ShapeDtypeStruct((NPAD, D), jnp.float32),
            jax.ShapeDtypeStruct((N,), jnp.int32),
        ],
        scratch_types=[
            pltpu.VMEM((TPW,), jnp.int32),
            pltpu.VMEM((TPW,), jnp.int32),
            pltpu.VMEM((16,), jnp.int32),
            pltpu.VMEM((TPW,), jnp.int32),
            pltpu.VMEM((2, 64), jnp.int32),
            pltpu.VMEM((64, D), jnp.float32),
            pltpu.SemaphoreType.DMA,
        ],
    )
    undispatch = pl.kernel(
        _undispatch_body,
        mesh=mesh,
        out_type=jax.ShapeDtypeStruct((S, B, D), jnp.float32),
        scratch_types=[
            pltpu.VMEM((2, 64), jnp.int32),
            pltpu.VMEM((64, D), jnp.float32),
            pltpu.SemaphoreType.DMA,
        ],
    )
    return dispatch, undispatch


def kernel(x, centroids, ln_scale, ln_bias, W1, b1, W2, b2):
    idx3, rank3, bc3 = _run_router(x, centroids)
    off16, te, valid = _routing_metadata(bc3)
    idx_flat = idx3.reshape(N)
    rank_flat = rank3.reshape(N)
    dispatch, undispatch = _sc_kernels()
    xs, slot = dispatch(x, idx_flat, rank_flat, off16)
    ys = _run_ffn(te, valid, xs, ln_scale, ln_bias, W1, b1, W2, b2)
    return undispatch(ys, slot)


# speedup vs baseline: 1.0905x; 1.0383x over previous
import functools

import jax
import jax.numpy as jnp
from jax import lax
from jax.experimental import pallas as pl
from jax.experimental.pallas import tpu as pltpu
from jax.experimental.pallas import tpu_sc as plsc

S, B, D, F, E = 2048, 2, 1024, 4096, 8
N = S * B
TBLK = 256
NBLK = N // TBLK
FB = F // 2
T = 512
NT = N // T + E
NPAD = NT * T

NC, NS = 2, 16
NW = NC * NS
TPW = N // NW


def _router_body(x_ref, c_ref, idx_ref, rank_ref, bc_ref, carry_ref):
    i = pl.program_id(0)

    @pl.when(i == 0)
    def _():
        carry_ref[...] = jnp.zeros_like(carry_ref)

    xb = jnp.concatenate([x_ref[:, 0, :], x_ref[:, 1, :]], axis=0)
    ct = c_ref[...]
    logits = lax.dot_general(xb, ct, (((1,), (1,)), ((), ())),
                             preferred_element_type=jnp.float32)
    idx = jnp.argmax(logits, axis=-1).astype(jnp.int32)
    lanes = lax.broadcasted_iota(jnp.int32, (TBLK, E), 1)
    oh = (idx[:, None] == lanes).astype(jnp.float32)
    r_i = lax.broadcasted_iota(jnp.int32, (TBLK, TBLK), 0)
    c_i = lax.broadcasted_iota(jnp.int32, (TBLK, TBLK), 1)
    tri = (c_i < r_i).astype(jnp.float32)
    cum = jnp.dot(tri, oh, preferred_element_type=jnp.float32)
    rank = jnp.sum((cum + carry_ref[...]) * oh, axis=1)
    bc = jnp.sum(oh, axis=0)
    idx_ref[...] = idx.reshape(1, 1, TBLK)
    rank_ref[...] = rank.astype(jnp.int32).reshape(1, 1, TBLK)
    bc_ref[...] = bc.reshape(1, 1, E)
    carry_ref[...] = carry_ref[...] + bc[None, :]


def _run_router(x, centroids):
    return pl.pallas_call(
        _router_body,
        grid=(NBLK,),
        in_specs=[
            pl.BlockSpec((TBLK // B, B, D), lambda i: (i, 0, 0)),
            pl.BlockSpec((E, D), lambda i: (0, 0)),
        ],
        out_specs=[
            pl.BlockSpec((1, 1, TBLK), lambda i: (i, 0, 0)),
            pl.BlockSpec((1, 1, TBLK), lambda i: (i, 0, 0)),
            pl.BlockSpec((1, 1, E), lambda i: (i, 0, 0)),
        ],
        out_shape=[
            jax.ShapeDtypeStruct((NBLK, 1, TBLK), jnp.int32),
            jax.ShapeDtypeStruct((NBLK, 1, TBLK), jnp.int32),
            jax.ShapeDtypeStruct((NBLK, 1, E), jnp.float32),
        ],
        scratch_shapes=[pltpu.VMEM((1, E), jnp.float32)],
        compiler_params=pltpu.CompilerParams(
            dimension_semantics=("arbitrary",)),
    )(x, centroids)


def _routing_metadata(bc3):
    bc = bc3.reshape(NBLK, E).astype(jnp.int32)
    counts = jnp.sum(bc, axis=0)
    pc = ((counts + T - 1) // T) * T
    poff = jnp.cumsum(pc) - pc
    off16 = jnp.zeros((16,), jnp.int32).at[:E].set(poff.astype(jnp.int32))
    tend = (poff + pc) // T
    kk = jnp.arange(NT, dtype=jnp.int32)
    te = jnp.minimum(
        jnp.sum((kk[:, None] >= tend[None, :]).astype(jnp.int32), axis=1),
        E - 1).astype(jnp.int32)
    valid = (kk < tend[E - 1]).astype(jnp.int32)
    return off16, te, valid


def _dispatch_body(x_hbm, idx_hbm, rank_hbm, off_hbm, xs_hbm, slot_hbm,
                   idxv, rankv, offv, slotf, slot2, rowbuf, sem):
    wid = lax.axis_index("s") * NC + lax.axis_index("c")
    base = wid * TPW
    s0 = (wid // B) * TPW
    bcol = wid % B
    pltpu.sync_copy(idx_hbm.at[pl.ds(base, TPW)], idxv)
    pltpu.sync_copy(rank_hbm.at[pl.ds(base, TPW)], rankv)
    pltpu.sync_copy(off_hbm, offv)
    for g in range(TPW // 16):
        e = idxv[pl.ds(g * 16, 16)]
        rk = rankv[pl.ds(g * 16, 16)]
        off = plsc.load_gather(offv, [e])
        sl = off + rk
        slotf[pl.ds(g * 16, 16)] = sl
        slot2[g // 4, pl.ds((g % 4) * 16, 16)] = sl
    pltpu.sync_copy(slotf, slot_hbm.at[pl.ds(base, TPW)])
    for h in range(2):
        pltpu.sync_copy(x_hbm.at[pl.ds(s0 + h * 64, 64), bcol], rowbuf)
        pltpu.async_copy(rowbuf, xs_hbm.at[slot2.at[h]], sem).wait()


def _ffn_a_body(te_ref, valid_ref, xs_ref, lns_ref, lnb_ref, w1_ref, b1_ref,
                w2_ref, b2_ref, ya_ref, lnx_ref):
    i = pl.program_id(0)

    @pl.when(valid_ref[i] != 0)
    def _():
        xv = xs_ref[...]
        mu = jnp.mean(xv, axis=1, keepdims=True)
        var = jnp.mean(jnp.square(xv - mu), axis=1, keepdims=True)
        lnx = ((xv - mu) * lax.rsqrt(var + 1e-5) * lns_ref[0, 0]
               + lnb_ref[0, 0])
        lnx_ref[...] = lnx
        h = jnp.maximum(
            jnp.dot(lnx, w1_ref[0], preferred_element_type=jnp.float32)
            + b1_ref[0, 0], 0.0)
        ya_ref[...] = xv + b2_ref[0, 0] + jnp.dot(
            h, w2_ref[0], preferred_element_type=jnp.float32)


def _ffn_b_body(te_ref, valid_ref, lnx_ref, ya_ref, w1_ref, b1_ref, w2_ref,
                out_ref):
    i = pl.program_id(0)

    @pl.when(valid_ref[i] != 0)
    def _():
        h = jnp.maximum(
            jnp.dot(lnx_ref[...], w1_ref[0],
                    preferred_element_type=jnp.float32) + b1_ref[0, 0], 0.0)
        out_ref[...] = ya_ref[...] + jnp.dot(
            h, w2_ref[0], preferred_element_type=jnp.float32)


def _run_ffn(te, valid, xs, ln_scale, ln_bias, W1, b1, W2, b2):
    b1r = b1.reshape(E, 1, F)
    ya, lnxs = pl.pallas_call(
        _ffn_a_body,
        grid_spec=pltpu.PrefetchScalarGridSpec(
            num_scalar_prefetch=2,
            grid=(NT,),
            in_specs=[
                pl.BlockSpec((T, D), lambda i, te, v: (i, 0)),
                pl.BlockSpec((1, 1, D), lambda i, te, v: (te[i], 0, 0)),
                pl.BlockSpec((1, 1, D), lambda i, te, v: (te[i], 0, 0)),
                pl.BlockSpec((1, D, FB), lambda i, te, v: (te[i], 0, 0)),
                pl.BlockSpec((1, 1, FB), lambda i, te, v: (te[i], 0, 0)),
                pl.BlockSpec((1, FB, D), lambda i, te, v: (te[i], 0, 0)),
                pl.BlockSpec((1, 1, D), lambda i, te, v: (te[i], 0, 0)),
            ],
            out_specs=[
                pl.BlockSpec((T, D), lambda i, te, v: (i, 0)),
                pl.BlockSpec((T, D), lambda i, te, v: (i, 0)),
            ],
        ),
        out_shape=[
            jax.ShapeDtypeStruct((NPAD, D), jnp.float32),
            jax.ShapeDtypeStruct((NPAD, D), jnp.float32),
        ],
        compiler_params=pltpu.CompilerParams(
            dimension_semantics=("arbitrary",)),
    )(te, valid, xs, ln_scale.reshape(E, 1, D), ln_bias.reshape(E, 1, D),
      W1, b1r, W2, b2.reshape(E, 1, D))
    return pl.pallas_call(
        _ffn_b_body,
        grid_spec=pltpu.PrefetchScalarGridSpec(
            num_scalar_prefetch=2,
            grid=(NT,),
            in_specs=[
                pl.BlockSpec((T, D), lambda i, te, v: (i, 0)),
                pl.BlockSpec((T, D), lambda i, te, v: (i, 0)),
                pl.BlockSpec((1, D, FB), lambda i, te, v: (te[i], 0, 1)),
                pl.BlockSpec((1, 1, FB), lambda i, te, v: (te[i], 0, 1)),
                pl.BlockSpec((1, FB, D), lambda i, te, v: (te[i], 1, 0)),
            ],
            out_specs=pl.BlockSpec((T, D), lambda i, te, v: (i, 0)),
        ),
        out_shape=jax.ShapeDtypeStruct((NPAD, D), jnp.float32),
        compiler_params=pltpu.CompilerParams(
            dimension_semantics=("arbitrary",)),
    )(te, valid, lnxs, ya, W1, b1r, W2)


def _undispatch_body(ys_hbm, slot_hbm, out_hbm, slot2, rowbuf, sem):
    wid = lax.axis_index("s") * NC + lax.axis_index("c")
    base = wid * TPW
    s0 = (wid // B) * TPW
    bcol = wid % B
    for h in range(2):
        pltpu.sync_copy(slot_hbm.at[pl.ds(base + h * 64, 64)], slot2.at[h])
        pltpu.async_copy(ys_hbm.at[slot2.at[h]], rowbuf, sem).wait()
        pltpu.sync_copy(rowbuf, out_hbm.at[pl.ds(s0 + h * 64, 64), bcol])


@functools.lru_cache(maxsize=1)
def _sc_kernels():
    mesh = plsc.VectorSubcoreMesh(core_axis_name="c", subcore_axis_name="s",
                                  num_cores=NC, num_subcores=NS)
    dispatch = pl.kernel(
        _dispatch_body,
        mesh=mesh,
        compiler_params=pltpu.CompilerParams(needs_layout_passes=False),
        out_type=[
            jax.ShapeDtypeStruct((NPAD, D), jnp.float32),
            jax.ShapeDtypeStruct((N,), jnp.int32),
        ],
        scratch_types=[
            pltpu.VMEM((TPW,), jnp.int32),
            pltpu.VMEM((TPW,), jnp.int32),
            pltpu.VMEM((16,), jnp.int32),
            pltpu.VMEM((TPW,), jnp.int32),
            pltpu.VMEM((2, 64), jnp.int32),
            pltpu.VMEM((64, D), jnp.float32),
            pltpu.SemaphoreType.DMA,
        ],
    )
    undispatch = pl.kernel(
        _undispatch_body,
        mesh=mesh,
        out_type=jax.ShapeDtypeStruct((S, B, D), jnp.float32),
        scratch_types=[
            pltpu.VMEM((2, 64), jnp.int32),
            pltpu.VMEM((64, D), jnp.float32),
            pltpu.SemaphoreType.DMA,
        ],
    )
    return dispatch, undispatch


def kernel(x, centroids, ln_scale, ln_bias, W1, b1, W2, b2):
    idx3, rank3, bc3 = _run_router(x, centroids)
    off16, te, valid = _routing_metadata(bc3)
    idx_flat = idx3.reshape(N)
    rank_flat = rank3.reshape(N)
    dispatch, undispatch = _sc_kernels()
    xs, slot = dispatch(x, idx_flat, rank_flat, off16)
    ys = _run_ffn(te, valid, xs, ln_scale, ln_bias, W1, b1, W2, b2)
    return undispatch(ys, slot)
